# baseline (speedup 1.0000x reference)
import functools

import jax
import jax.numpy as jnp
from jax import lax
from jax.experimental import pallas as pl
from jax.experimental.pallas import tpu as pltpu
from jax.experimental.pallas import tpu_sc as plsc

N_NODES = 10000
N_EDGES = 320000
D = 128

NC = 2
NS = 16
NW = NC * NS

NP = 10240
CE = 64
EP = 327680
CHUNKS = EP // (NW * CE)
NB = 4
STAGES = 4
CPS = CHUNKS // STAGES
GPS = CPS // NB
ROWS_PER_TILE = NP // NS
CNT_ROWS = NP // 16


def _sc_aggregate(x, src3, dst3):
    mesh = plsc.VectorSubcoreMesh(core_axis_name="c", subcore_axis_name="s")

    @functools.partial(
        pl.kernel,
        out_type=(
            jax.ShapeDtypeStruct((NC, NP, D), jnp.float32),
            jax.ShapeDtypeStruct((NC, NP), jnp.float32),
        ),
        mesh=mesh,
        scratch_types=[
            pltpu.VMEM((CPS, CE), jnp.int32),
            pltpu.VMEM((CPS, CE), jnp.int32),
            pltpu.VMEM((NB, CE, D), jnp.float32),
            pltpu.VMEM((CE,), jnp.float32),
            pltpu.VMEM((ROWS_PER_TILE,), jnp.float32),
            pltpu.VMEM_SHARED((NP, D), jnp.float32),
            pltpu.VMEM_SHARED((NP,), jnp.float32),
        ] + [pltpu.SemaphoreType.DMA] * (3 * NB),
    )
    def k(x_hbm, src_hbm, dst_hbm, agg_out, cnt_out,
          idx_s, idx_d, rows, ones_v, zcnt, agg_sh, cnt_sh, *sems):
        gsem = sems[:NB]
        ssem = sems[NB:2 * NB]
        csem = sems[2 * NB:]
        cid = lax.axis_index("c")
        sid = lax.axis_index("s")
        wid = sid * NC + cid

        zf = jnp.zeros((16,), jnp.float32)
        ones16 = jnp.ones((16,), jnp.float32)

        def zero_rows(i, _):
            r = i >> 3
            c = (i & 7) * 16
            rows[0, r, pl.ds(c, 16)] = zf
            return _
        lax.fori_loop(0, CE * (D // 16), zero_rows, None)

        def fill_small(i, _):
            ones_v[pl.ds(i * 16, 16)] = ones16
            return _
        lax.fori_loop(0, CE // 16, fill_small, None)

        def zero_zcnt(i, _):
            zcnt[pl.ds(i * 16, 16)] = zf
            return _
        lax.fori_loop(0, ROWS_PER_TILE // 16, zero_zcnt, None)

        base = sid * ROWS_PER_TILE
        for j in range(ROWS_PER_TILE // CE):
            pltpu.sync_copy(rows.at[0], agg_sh.at[pl.ds(base + j * CE, CE)])
        pltpu.sync_copy(zcnt, cnt_sh.at[pl.ds(base, ROWS_PER_TILE)])

        plsc.subcore_barrier()

        for st in range(STAGES):
            pltpu.sync_copy(src_hbm.at[wid, pl.ds(st * CPS, CPS)], idx_s)
            pltpu.sync_copy(dst_hbm.at[wid, pl.ds(st * CPS, CPS)], idx_d)
            for b in range(NB):
                pltpu.async_copy(x_hbm.at[idx_s.at[b]], rows.at[b], gsem[b])

            def group_body(g, _):
                for p in range(NB // 2):
                    b0, b1 = 2 * p, 2 * p + 1
                    j0 = g * NB + b0
                    j1 = g * NB + b1
                    pltpu.make_async_copy(x_hbm.at[idx_s.at[j0]], rows.at[b0],
                                          gsem[b0]).wait()
                    s0 = pltpu.async_copy(rows.at[b0], agg_sh.at[idx_d.at[j0]],
                                          ssem[b0], add=True)
                    c0 = pltpu.async_copy(ones_v, cnt_sh.at[idx_d.at[j0]],
                                          csem[b0], add=True)
                    pltpu.make_async_copy(x_hbm.at[idx_s.at[j1]], rows.at[b1],
                                          gsem[b1]).wait()
                    s1 = pltpu.async_copy(rows.at[b1], agg_sh.at[idx_d.at[j1]],
                                          ssem[b1], add=True)
                    c1 = pltpu.async_copy(ones_v, cnt_sh.at[idx_d.at[j1]],
                                          csem[b1], add=True)
                    s0.wait()
                    c0.wait()

                    @pl.when(g < GPS - 1)
                    def _():
                        pltpu.async_copy(x_hbm.at[idx_s.at[j0 + NB]],
                                         rows.at[b0], gsem[b0])
                    s1.wait()
                    c1.wait()

                    @pl.when(g < GPS - 1)
                    def _():
                        pltpu.async_copy(x_hbm.at[idx_s.at[j1 + NB]],
                                         rows.at[b1], gsem[b1])
                return _
            lax.fori_loop(0, GPS, group_body, None)

        plsc.subcore_barrier()

        pltpu.sync_copy(agg_sh.at[pl.ds(base, ROWS_PER_TILE)],
                        agg_out.at[cid, pl.ds(base, ROWS_PER_TILE)])
        pltpu.sync_copy(cnt_sh.at[pl.ds(base, ROWS_PER_TILE)],
                        cnt_out.at[cid, pl.ds(base, ROWS_PER_TILE)])

    return k(x, src3, dst3)


def _tc_update(agg, cnt2, x, Wt, Bt):
    R = 1000

    def body(agg_ref, cnt_ref, x_ref, wt_ref, bt_ref, o_ref):
        a = agg_ref[0] + agg_ref[1]
        c = cnt_ref[0] + cnt_ref[1]
        denom = jnp.where(c == 0.0, 1.0, c)
        msg = a / denom
        o_ref[...] = (
            jnp.dot(msg, wt_ref[...], preferred_element_type=jnp.float32)
            + jnp.dot(x_ref[...], bt_ref[...], preferred_element_type=jnp.float32)
        )

    return pl.pallas_call(
        body,
        grid=(N_NODES // R,),
        in_specs=[
            pl.BlockSpec((NC, R, D), lambda i: (0, i, 0)),
            pl.BlockSpec((NC, R, 1), lambda i: (0, i, 0)),
            pl.BlockSpec((R, D), lambda i: (i, 0)),
            pl.BlockSpec((D, D), lambda i: (0, 0)),
            pl.BlockSpec((D, D), lambda i: (0, 0)),
        ],
        out_specs=pl.BlockSpec((R, D), lambda i: (i, 0)),
        out_shape=jax.ShapeDtypeStruct((N_NODES, D), jnp.float32),
    )(agg, cnt2, x, Wt, Bt)


def kernel(x, edge_index, W, B):
    src = edge_index[0].astype(jnp.int32)
    dst = edge_index[1].astype(jnp.int32)
    pad = EP - N_EDGES
    pad_iota = lax.iota(jnp.int32, pad)
    src_p = jnp.concatenate([src, pad_iota % N_NODES])
    dst_p = jnp.concatenate([dst, N_NODES + pad_iota % (NP - N_NODES)])
    src3 = src_p.reshape(NW, CHUNKS, CE)
    dst3 = dst_p.reshape(NW, CHUNKS, CE)

    agg, cnt = _sc_aggregate(x, src3, dst3)
    cnt2 = cnt.reshape(NC, NP, 1)
    return _tc_update(agg, cnt2, x, W.T, B.T)


# speedup vs baseline: 1.0267x; 1.0267x over previous
import functools

import jax
import jax.numpy as jnp
from jax import lax
from jax.experimental import pallas as pl
from jax.experimental.pallas import tpu as pltpu
from jax.experimental.pallas import tpu_sc as plsc

N_NODES = 10000
N_EDGES = 320000
D = 128

NC = 2
NS = 16
NW = NC * NS

NP = 10240
CE = 40
CHUNKS = N_EDGES // (NW * CE)
NB = 5
STAGES = 5
CPS = CHUNKS // STAGES
GPS = CPS // NB
EPT = N_EDGES // NW
ROWS_PER_TILE = NP // NS
CNT_ROWS = NP // 16


def _sc_aggregate(x, src3, dst3):
    mesh = plsc.VectorSubcoreMesh(core_axis_name="c", subcore_axis_name="s")

    @functools.partial(
        pl.kernel,
        out_type=(
            jax.ShapeDtypeStruct((NC, NP, D), jnp.float32),
            jax.ShapeDtypeStruct((NC, NP), jnp.float32),
        ),
        mesh=mesh,
        scratch_types=[
            pltpu.VMEM((EPT,), jnp.int32),
            pltpu.VMEM((CPS, CE), jnp.int32),
            pltpu.VMEM((NB, CE, D), jnp.float32),
            pltpu.VMEM((CE,), jnp.float32),
            pltpu.VMEM((ROWS_PER_TILE,), jnp.float32),
            pltpu.VMEM_SHARED((NP, D), jnp.float32),
            pltpu.VMEM_SHARED((NP,), jnp.float32),
        ] + [pltpu.SemaphoreType.DMA] * (3 * NB),
    )
    def k(x_hbm, src_hbm, dst_hbm, agg_out, cnt_out,
          idx_s, idx_d, rows, ones_v, zcnt, agg_sh, cnt_sh, *sems):
        gsem = sems[:NB]
        ssem = sems[NB:2 * NB]
        csem = sems[2 * NB:]
        cid = lax.axis_index("c")
        sid = lax.axis_index("s")
        wid = sid * NC + cid

        zf = jnp.zeros((16,), jnp.float32)
        ones16 = jnp.ones((16,), jnp.float32)

        def zero_rows(i, _):
            r = i >> 3
            c = (i & 7) * 16
            rows[0, r, pl.ds(c, 16)] = zf
            return _
        lax.fori_loop(0, CE * (D // 16), zero_rows, None)

        def fill_small(i, _):
            ones_v[pl.ds(i * 16, 16)] = ones16
            return _
        lax.fori_loop(0, CE // 16, fill_small, None)

        def zero_zcnt(i, _):
            zcnt[pl.ds(i * 16, 16)] = zf
            return _
        lax.fori_loop(0, ROWS_PER_TILE // 16, zero_zcnt, None)

        base = sid * ROWS_PER_TILE
        for j in range(ROWS_PER_TILE // CE):
            pltpu.sync_copy(rows.at[0], agg_sh.at[pl.ds(base + j * CE, CE)])
        pltpu.sync_copy(zcnt, cnt_sh.at[pl.ds(base, ROWS_PER_TILE)])

        pltpu.sync_copy(src_hbm.at[wid], idx_s)

        plsc.subcore_barrier()

        for st in range(STAGES):
            pltpu.sync_copy(dst_hbm.at[wid, st], idx_d)
            for b in range(NB):
                jj = st * CPS + b
                pltpu.async_copy(x_hbm.at[idx_s.at[pl.ds(jj * CE, CE)]],
                                 rows.at[b], gsem[b])

            def group_body(g, _):
                for b in range(NB):
                    j = g * NB + b
                    jj = st * CPS + j
                    pltpu.make_async_copy(
                        x_hbm.at[idx_s.at[pl.ds(jj * CE, CE)]], rows.at[b],
                        gsem[b]).wait()
                    sd = pltpu.async_copy(rows.at[b], agg_sh.at[idx_d.at[j]],
                                          ssem[b], add=True)
                    cd = pltpu.async_copy(ones_v, cnt_sh.at[idx_d.at[j]],
                                          csem[b], add=True)
                    sd.wait()
                    cd.wait()

                    @pl.when(g < GPS - 1)
                    def _():
                        pltpu.async_copy(
                            x_hbm.at[idx_s.at[pl.ds((jj + NB) * CE, CE)]],
                            rows.at[b], gsem[b])
                return _
            lax.fori_loop(0, GPS, group_body, None)

        plsc.subcore_barrier()

        pltpu.sync_copy(agg_sh.at[pl.ds(base, ROWS_PER_TILE)],
                        agg_out.at[cid, pl.ds(base, ROWS_PER_TILE)])
        pltpu.sync_copy(cnt_sh.at[pl.ds(base, ROWS_PER_TILE)],
                        cnt_out.at[cid, pl.ds(base, ROWS_PER_TILE)])

    return k(x, src3, dst3)


def _tc_update(agg, cnt2, x, Wt, Bt):
    R = 1000

    def body(agg_ref, cnt_ref, x_ref, wt_ref, bt_ref, o_ref):
        a = agg_ref[0] + agg_ref[1]
        c = cnt_ref[0] + cnt_ref[1]
        denom = jnp.where(c == 0.0, 1.0, c)
        msg = a / denom
        o_ref[...] = (
            jnp.dot(msg, wt_ref[...], preferred_element_type=jnp.float32)
            + jnp.dot(x_ref[...], bt_ref[...], preferred_element_type=jnp.float32)
        )

    return pl.pallas_call(
        body,
        grid=(N_NODES // R,),
        in_specs=[
            pl.BlockSpec((NC, R, D), lambda i: (0, i, 0)),
            pl.BlockSpec((NC, R, 1), lambda i: (0, i, 0)),
            pl.BlockSpec((R, D), lambda i: (i, 0)),
            pl.BlockSpec((D, D), lambda i: (0, 0)),
            pl.BlockSpec((D, D), lambda i: (0, 0)),
        ],
        out_specs=pl.BlockSpec((R, D), lambda i: (i, 0)),
        out_shape=jax.ShapeDtypeStruct((N_NODES, D), jnp.float32),
    )(agg, cnt2, x, Wt, Bt)


def kernel(x, edge_index, W, B):
    src = edge_index[0].astype(jnp.int32)
    dst = edge_index[1].astype(jnp.int32)
    src3 = src.reshape(NW, EPT)
    dst3 = dst.reshape(NW, STAGES, CPS, CE)

    agg, cnt = _sc_aggregate(x, src3, dst3)
    cnt2 = cnt.reshape(NC, NP, 1)
    return _tc_update(agg, cnt2, x, W.T, B.T)
